# baseline (speedup 1.0000x reference)
import functools

import jax
import jax.numpy as jnp
from jax import lax
from jax.experimental import pallas as pl
from jax.experimental.pallas import tpu as pltpu
from jax.experimental.pallas import tpu_sc as plsc

VOCAB = 1000000
EMB = 64
BATCH = 16384
HIST = 50

NW = 32
BBLK = 128
NC = BATCH // BBLK
UNITS = HIST * NC
UNITS_PER_W = UNITS // NW
L = 16


def _make_kernel():
    mesh = plsc.VectorSubcoreMesh(core_axis_name="c", subcore_axis_name="s")

    @functools.partial(
        pl.kernel,
        mesh=mesh,
        compiler_params=pltpu.CompilerParams(needs_layout_passes=False),
        out_type=jax.ShapeDtypeStruct((HIST, EMB, BATCH), jnp.float32),
        scratch_types=[
            pltpu.VMEM((2, BBLK), jnp.int32),
            pltpu.VMEM((2, BBLK), jnp.int32),
            pltpu.VMEM((2, BBLK, 128), jnp.float32),
            pltpu.VMEM((2, EMB, BBLK), jnp.float32),
            pltpu.SemaphoreType.DMA,
            pltpu.SemaphoreType.DMA,
            pltpu.SemaphoreType.DMA,
            pltpu.SemaphoreType.DMA,
        ],
    )
    def emb_kernel(items_t, table2, out_hbm, idx_v, idxu_v, rows_v, obuf_v,
                   gsem0, gsem1, wsem0, wsem1):
        wid = lax.axis_index("s") * 2 + lax.axis_index("c")
        ubase = wid * UNITS_PER_W
        gsems = (gsem0, gsem1)
        wsems = (wsem0, wsem1)

        def stage_and_fire(t, b):
            uid = ubase + t
            h = uid // NC
            c = uid % NC
            pltpu.sync_copy(items_t.at[h, pl.ds(c * BBLK, BBLK)],
                            idx_v.at[b])
            for g in range(BBLK // L):
                v16 = idx_v[b, pl.ds(g * L, L)]
                idxu_v[b, pl.ds(g * L, L)] = v16 >> 1
            pltpu.async_copy(table2.at[idxu_v.at[b]], rows_v.at[b],
                             gsems[b])

        def drain_gather(b):
            pltpu.make_async_copy(table2.at[idxu_v.at[b]], rows_v.at[b],
                                  gsems[b]).wait()

        def transpose_unit(t, b):
            uid = ubase + t
            h = uid // NC
            c = uid % NC
            rows_b = rows_v.at[b]
            colbases = []
            rowidxs = []
            for g in range(BBLK // L):
                v16 = idx_v[b, pl.ds(g * L, L)]
                colbases.append((v16 & 1) << 6)
                rowidxs.append(lax.iota(jnp.int32, L) + (g * L))
            obuf_b = obuf_v.at[b]

            @plsc.parallel_loop(0, EMB, step=1, unroll=4)
            def _(e):
                for g in range(BBLK // L):
                    val = plsc.load_gather(rows_b,
                                           [rowidxs[g], colbases[g] + e])
                    obuf_b[e, pl.ds(g * L, L)] = val
            pltpu.async_copy(
                obuf_v.at[b],
                out_hbm.at[h, :, pl.ds(c * BBLK, BBLK)],
                wsems[b],
            )

        def drain_write(t, b):
            uid = ubase + t
            h = uid // NC
            c = uid % NC
            pltpu.make_async_copy(
                obuf_v.at[b],
                out_hbm.at[h, :, pl.ds(c * BBLK, BBLK)],
                wsems[b],
            ).wait()

        stage_and_fire(0, 0)

        def superstep(s):
            for b in range(2):
                t = s * 2 + b
                nb = 1 - b

                @pl.when(t > 0)
                def _():
                    drain_write(t - 1, nb)

                @pl.when(t + 1 < UNITS_PER_W)
                def _():
                    stage_and_fire(t + 1, nb)

                drain_gather(b)
                transpose_unit(t, b)

        pl.loop(0, UNITS_PER_W // 2)(superstep)
        drain_write(UNITS_PER_W - 1, 1)

    return emb_kernel


_emb_kernel = _make_kernel()


def kernel(items, weight):
    items_t = items.astype(jnp.int32).T
    table2 = weight.reshape(VOCAB // 2, 128)
    out = _emb_kernel(items_t, table2)
    return out.transpose(2, 0, 1)


# speedup vs baseline: 1.5461x; 1.5461x over previous
import functools

import jax
import jax.numpy as jnp
from jax import lax
from jax.experimental import pallas as pl
from jax.experimental.pallas import tpu as pltpu
from jax.experimental.pallas import tpu_sc as plsc

VOCAB = 1000000
EMB = 64
BATCH = 16384
HIST = 50

NW = 32
BBLK = 128
NC = BATCH // BBLK
UNITS = HIST * NC
UNITS_PER_W = UNITS // NW
L = 16


def _make_kernel():
    mesh = plsc.VectorSubcoreMesh(core_axis_name="c", subcore_axis_name="s")

    @functools.partial(
        pl.kernel,
        mesh=mesh,
        compiler_params=pltpu.CompilerParams(needs_layout_passes=False),
        out_type=jax.ShapeDtypeStruct((HIST, EMB, BATCH), jnp.float32),
        scratch_types=[
            pltpu.VMEM((2, BBLK), jnp.int32),
            pltpu.VMEM((2, BBLK), jnp.int32),
            pltpu.VMEM((2, BBLK, 128), jnp.float32),
            pltpu.VMEM((2, EMB, BBLK), jnp.float32),
            pltpu.SemaphoreType.DMA,
            pltpu.SemaphoreType.DMA,
            pltpu.SemaphoreType.DMA,
            pltpu.SemaphoreType.DMA,
        ],
    )
    def emb_kernel(items_t, table2, out_hbm, idx_v, idxu_v, rows_v, obuf_v,
                   gsem0, gsem1, wsem0, wsem1):
        wid = lax.axis_index("s") * 2 + lax.axis_index("c")
        ubase = wid * UNITS_PER_W
        gsems = (gsem0, gsem1)
        wsems = (wsem0, wsem1)

        def stage_and_fire(t, b):
            uid = ubase + t
            h = uid // NC
            c = uid % NC
            pltpu.sync_copy(items_t.at[h, pl.ds(c * BBLK, BBLK)],
                            idx_v.at[b])
            for g in range(BBLK // L):
                v16 = idx_v[b, pl.ds(g * L, L)]
                idxu_v[b, pl.ds(g * L, L)] = v16 >> 1
            pltpu.async_copy(table2.at[idxu_v.at[b]], rows_v.at[b],
                             gsems[b])

        def drain_gather(b):
            pltpu.make_async_copy(table2.at[idxu_v.at[b]], rows_v.at[b],
                                  gsems[b]).wait()

        def transpose_unit(t, b):
            uid = ubase + t
            h = uid // NC
            c = uid % NC
            rows_b = rows_v.at[b]
            obuf_b = obuf_v.at[b]
            iota = lax.iota(jnp.int32, L)
            for J in range(BBLK // L):
                v16 = idx_v[b, pl.ds(J * L, L)]
                odd64 = (v16 & 1) << 6
                jrow = iota + (J * L)

                @plsc.parallel_loop(0, L, step=1, unroll=2)
                def _(k):
                    pat = (iota + k) & (L - 1)
                    for E in range(EMB // L):
                        erow = pat + (E * L)
                        val = plsc.load_gather(rows_b, [jrow, odd64 + erow])
                        plsc.store_scatter(obuf_b, [erow, jrow], val)
            pltpu.async_copy(
                obuf_v.at[b],
                out_hbm.at[h, :, pl.ds(c * BBLK, BBLK)],
                wsems[b],
            )

        def drain_write(t, b):
            uid = ubase + t
            h = uid // NC
            c = uid % NC
            pltpu.make_async_copy(
                obuf_v.at[b],
                out_hbm.at[h, :, pl.ds(c * BBLK, BBLK)],
                wsems[b],
            ).wait()

        stage_and_fire(0, 0)

        def superstep(s):
            for b in range(2):
                t = s * 2 + b
                nb = 1 - b

                @pl.when(t > 0)
                def _():
                    drain_write(t - 1, nb)

                @pl.when(t + 1 < UNITS_PER_W)
                def _():
                    stage_and_fire(t + 1, nb)

                drain_gather(b)
                transpose_unit(t, b)

        pl.loop(0, UNITS_PER_W // 2)(superstep)
        drain_write(UNITS_PER_W - 1, 1)

    return emb_kernel


_emb_kernel = _make_kernel()


def kernel(items, weight):
    items_t = items.astype(jnp.int32).T
    table2 = weight.reshape(VOCAB // 2, 128)
    out = _emb_kernel(items_t, table2)
    return out.transpose(2, 0, 1)


# speedup vs baseline: 2.3219x; 1.5018x over previous
import functools

import jax
import jax.numpy as jnp
from jax import lax
from jax.experimental import pallas as pl
from jax.experimental.pallas import tpu as pltpu
from jax.experimental.pallas import tpu_sc as plsc

VOCAB = 1000000
EMB = 64
BATCH = 16384
HIST = 50

NW = 32
BBLK = 128
NC = BATCH // BBLK
UNITS = HIST * NC
UNITS_PER_W = UNITS // NW
L = 16


NFULL = VOCAB // 128
NTAIL = (VOCAB - NFULL * 128) // 2
ITERS1 = -(-NFULL // NW)


def _make_detranspose():
    mesh = plsc.VectorSubcoreMesh(core_axis_name="c", subcore_axis_name="s")

    @functools.partial(
        pl.kernel,
        mesh=mesh,
        compiler_params=pltpu.CompilerParams(needs_layout_passes=False),
        out_type=jax.ShapeDtypeStruct((VOCAB // 2, 128), jnp.float32),
        scratch_types=[
            pltpu.VMEM((2, EMB, 128), jnp.float32),
            pltpu.VMEM((2, EMB, 128), jnp.float32),
            pltpu.VMEM((NTAIL, 128), jnp.float32),
            pltpu.SemaphoreType.DMA,
            pltpu.SemaphoreType.DMA,
            pltpu.SemaphoreType.DMA,
            pltpu.SemaphoreType.DMA,
        ],
    )
    def detr_kernel(wt_hbm, tail_hbm, w2_hbm, ibuf, obuf, tbuf,
                    gsem0, gsem1, wsem0, wsem1):
        wid = lax.axis_index("s") * 2 + lax.axis_index("c")
        gsems = (gsem0, gsem1)
        wsems = (wsem0, wsem1)
        iota = lax.iota(jnp.int32, L)

        @pl.when(wid == NW - 1)
        def _():
            pltpu.sync_copy(tail_hbm, tbuf)
            pltpu.sync_copy(tbuf, w2_hbm.at[pl.ds(NFULL * 64, NTAIL)])

        def uid_of(t):
            return t * NW + wid

        def fire_in(t, b):
            c = uid_of(t)
            pltpu.async_copy(wt_hbm.at[:, pl.ds(c * 128, 128)],
                             ibuf.at[b], gsems[b])

        def drain_in(t, b):
            c = uid_of(t)
            pltpu.make_async_copy(wt_hbm.at[:, pl.ds(c * 128, 128)],
                                  ibuf.at[b], gsems[b]).wait()

        def transpose_block(t, b):
            ibuf_b = ibuf.at[b]
            obuf_b = obuf.at[b]
            for J in range(128 // L):
                jrow = iota + (J * L)
                jrow2 = jrow >> 1
                oddj = (jrow & 1) << 6

                @plsc.parallel_loop(0, L, step=1, unroll=2)
                def _(k):
                    pat = (iota + k) & (L - 1)
                    for E in range(EMB // L):
                        erow = pat + (E * L)
                        val = plsc.load_gather(ibuf_b, [erow, jrow])
                        plsc.store_scatter(obuf_b, [jrow2, oddj + erow], val)

        def fire_out(t, b):
            c = uid_of(t)
            pltpu.async_copy(obuf.at[b], w2_hbm.at[pl.ds(c * 64, 64)],
                             wsems[b])

        def drain_out(t, b):
            c = uid_of(t)
            pltpu.make_async_copy(obuf.at[b], w2_hbm.at[pl.ds(c * 64, 64)],
                                  wsems[b]).wait()

        fire_in(0, 0)

        def superstep(s):
            for b in range(2):
                t = s * 2 + b
                nb = 1 - b

                @pl.when(t > 0)
                def _():
                    drain_out(t - 1, nb)

                @pl.when(uid_of(t + 1) < NFULL)
                def _():
                    fire_in(t + 1, nb)

                @pl.when(uid_of(t) < NFULL)
                def _():
                    drain_in(t, b)
                    transpose_block(t, b)
                    fire_out(t, b)

        pl.loop(0, ITERS1 // 2)(superstep)

        t_last = ITERS1 - 1
        drain_out(t_last - 1, 1)

        @pl.when(uid_of(t_last) < NFULL)
        def _():
            drain_in(t_last, 0)
            transpose_block(t_last, 0)
            fire_out(t_last, 0)
            drain_out(t_last, 0)

    return detr_kernel


def _make_kernel():
    mesh = plsc.VectorSubcoreMesh(core_axis_name="c", subcore_axis_name="s")

    @functools.partial(
        pl.kernel,
        mesh=mesh,
        compiler_params=pltpu.CompilerParams(needs_layout_passes=False),
        out_type=jax.ShapeDtypeStruct((HIST, EMB, BATCH), jnp.float32),
        scratch_types=[
            pltpu.VMEM((2, BBLK), jnp.int32),
            pltpu.VMEM((2, BBLK), jnp.int32),
            pltpu.VMEM((2, BBLK, 128), jnp.float32),
            pltpu.VMEM((2, EMB, BBLK), jnp.float32),
            pltpu.SemaphoreType.DMA,
            pltpu.SemaphoreType.DMA,
            pltpu.SemaphoreType.DMA,
            pltpu.SemaphoreType.DMA,
        ],
    )
    def emb_kernel(items_t, table2, out_hbm, idx_v, idxu_v, rows_v, obuf_v,
                   gsem0, gsem1, wsem0, wsem1):
        wid = lax.axis_index("s") * 2 + lax.axis_index("c")
        ubase = wid * UNITS_PER_W
        gsems = (gsem0, gsem1)
        wsems = (wsem0, wsem1)

        def stage_and_fire(t, b):
            uid = ubase + t
            h = uid // NC
            c = uid % NC
            pltpu.sync_copy(items_t.at[h, pl.ds(c * BBLK, BBLK)],
                            idx_v.at[b])
            for g in range(BBLK // L):
                v16 = idx_v[b, pl.ds(g * L, L)]
                idxu_v[b, pl.ds(g * L, L)] = v16 >> 1
            pltpu.async_copy(table2.at[idxu_v.at[b]], rows_v.at[b],
                             gsems[b])

        def drain_gather(b):
            pltpu.make_async_copy(table2.at[idxu_v.at[b]], rows_v.at[b],
                                  gsems[b]).wait()

        def transpose_unit(t, b):
            uid = ubase + t
            h = uid // NC
            c = uid % NC
            rows_b = rows_v.at[b]
            obuf_b = obuf_v.at[b]
            iota = lax.iota(jnp.int32, L)
            for J in range(BBLK // L):
                v16 = idx_v[b, pl.ds(J * L, L)]
                odd64 = (v16 & 1) << 6
                jrow = iota + (J * L)

                @plsc.parallel_loop(0, L, step=1, unroll=2)
                def _(k):
                    pat = (iota + k) & (L - 1)
                    for E in range(EMB // L):
                        erow = pat + (E * L)
                        val = plsc.load_gather(rows_b, [jrow, odd64 + erow])
                        plsc.store_scatter(obuf_b, [erow, jrow], val)
            pltpu.async_copy(
                obuf_v.at[b],
                out_hbm.at[h, :, pl.ds(c * BBLK, BBLK)],
                wsems[b],
            )

        def drain_write(t, b):
            uid = ubase + t
            h = uid // NC
            c = uid % NC
            pltpu.make_async_copy(
                obuf_v.at[b],
                out_hbm.at[h, :, pl.ds(c * BBLK, BBLK)],
                wsems[b],
            ).wait()

        stage_and_fire(0, 0)

        def superstep(s):
            for b in range(2):
                t = s * 2 + b
                nb = 1 - b

                @pl.when(t > 0)
                def _():
                    drain_write(t - 1, nb)

                @pl.when(t + 1 < UNITS_PER_W)
                def _():
                    stage_and_fire(t + 1, nb)

                drain_gather(b)
                transpose_unit(t, b)

        pl.loop(0, UNITS_PER_W // 2)(superstep)
        drain_write(UNITS_PER_W - 1, 1)

    return emb_kernel


_emb_kernel = _make_kernel()
_detr_kernel = _make_detranspose()


def kernel(items, weight):
    items_t = items.astype(jnp.int32).T
    tail = weight[NFULL * 128:, :].reshape(NTAIL, 128)
    table2 = _detr_kernel(weight.T, tail)
    out = _emb_kernel(items_t, table2)
    return out.transpose(2, 0, 1)
